# baseline (speedup 1.0000x reference)
import functools

import jax
import jax.numpy as jnp
from jax import lax
from jax.experimental import pallas as pl
from jax.experimental.pallas import tpu as pltpu
from jax.experimental.pallas import tpu_sc as plsc

NUM_NODES = 1000000
EMBED_DIM = 64
BATCH = 16384
HIST = 50

_HP = 64
_HG = 56
_BPC = 16
_IR_PC = _BPC * _HP // 128


def _make_gather(nw: int):
    b_per_w = BATCH // nw
    n_chunks = b_per_w // _BPC
    n_pairs = n_chunks // 2
    idx_rows_per_w = b_per_w * _HP // 128
    mesh = plsc.VectorSubcoreMesh(core_axis_name="c", subcore_axis_name="s")

    @functools.partial(
        pl.kernel,
        out_type=jax.ShapeDtypeStruct((BATCH, HIST, EMBED_DIM), jnp.float32),
        mesh=mesh,
        scratch_types=[
            pltpu.VMEM((_IR_PC, 128), jnp.int32),
            pltpu.VMEM((_IR_PC, 128), jnp.int32),
            pltpu.VMEM((_BPC, _HG, EMBED_DIM), jnp.float32),
            pltpu.VMEM((_BPC, _HG, EMBED_DIM), jnp.float32),
            pltpu.SemaphoreType.DMA,
            pltpu.SemaphoreType.DMA,
            pltpu.SemaphoreType.DMA,
            pltpu.SemaphoreType.DMA,
            pltpu.SemaphoreType.DMA,
            pltpu.SemaphoreType.DMA,
        ],
        compiler_params=pltpu.CompilerParams(use_tc_tiling_on_sc=False),
    )
    def gather_kernel(idx_hbm, table_hbm, out_hbm, ibuf_a, ibuf_b,
                      stage_a, stage_b, isem_a, isem_b, gsem_a, gsem_b,
                      osem_a, osem_b):
        nc = lax.axis_size("c")
        wid = lax.axis_index("s") * nc + lax.axis_index("c")
        b_base = wid * b_per_w
        idx_base = wid * idx_rows_per_w

        def idx_copy(chunk, ibuf, isem):
            row0 = pl.multiple_of(idx_base + chunk * _IR_PC, 8)
            return pltpu.make_async_copy(
                idx_hbm.at[pl.ds(row0, _IR_PC), :], ibuf, isem)

        def gathers(ibuf, stage, gsem):
            return [
                pltpu.make_async_copy(
                    table_hbm.at[ibuf.at[bb // 2, pl.ds((bb % 2) * _HP, _HG)]],
                    stage.at[bb],
                    gsem,
                )
                for bb in range(_BPC)
            ]

        def writeback(chunk, stage, osem):
            b0 = pl.multiple_of(b_base + chunk * _BPC, _BPC)
            return pltpu.make_async_copy(
                stage.at[:, pl.ds(0, HIST)], out_hbm.at[pl.ds(b0, _BPC)], osem)

        idx_copy(0, ibuf_a, isem_a).start()
        idx_copy(1, ibuf_b, isem_b).start()
        idx_copy(0, ibuf_a, isem_a).wait()
        for g in gathers(ibuf_a, stage_a, gsem_a):
            g.start()

        def body(i, carry):
            c0 = 2 * i
            for g in gathers(ibuf_a, stage_a, gsem_a):
                g.wait()

            @pl.when(i + 1 < n_pairs)
            def _():
                idx_copy(c0 + 2, ibuf_a, isem_a).start()

            @pl.when(i >= 1)
            def _():
                writeback(0, stage_b, osem_b).wait()

            idx_copy(0, ibuf_b, isem_b).wait()
            for g in gathers(ibuf_b, stage_b, gsem_b):
                g.start()
            writeback(c0, stage_a, osem_a).start()

            for g in gathers(ibuf_b, stage_b, gsem_b):
                g.wait()

            @pl.when(i + 1 < n_pairs)
            def _():
                idx_copy(c0 + 3, ibuf_b, isem_b).start()

            writeback(0, stage_a, osem_a).wait()

            @pl.when(i + 1 < n_pairs)
            def _():
                idx_copy(0, ibuf_a, isem_a).wait()
                for g in gathers(ibuf_a, stage_a, gsem_a):
                    g.start()

            writeback(c0 + 1, stage_b, osem_b).start()
            return carry

        lax.fori_loop(0, n_pairs, body, 0)
        writeback(0, stage_b, osem_b).wait()

    return gather_kernel


def kernel(nodes, emb_weight):
    info = plsc.get_sparse_core_info()
    nw = info.num_cores * info.num_subcores
    idx2d = jnp.pad(nodes, ((0, 0), (0, _HP - HIST))).reshape(
        BATCH * _HP // 128, 128)
    return _make_gather(nw)(idx2d, emb_weight)
